# baseline (speedup 1.0000x reference)
import functools

import jax
import jax.numpy as jnp
from jax import lax
from jax.experimental import pallas as pl
from jax.experimental.pallas import tpu as pltpu
from jax.experimental.pallas import tpu_sc as plsc

N = 10000
E = 160000
D = 128
DOUT = 256
NC = 2
NS = 16
L = 16

CHUNK = 128
NCHUNKS = E // CHUNK
NFULL = NCHUNKS // NS
EXTRA = NCHUNKS % NS
EPT = E // NS

NPAD = 10240
RPT = NPAD // NS
ZR = 32
TAIL = N - (NS - 1) * RPT

_sc_mesh = plsc.VectorSubcoreMesh(core_axis_name="c", subcore_axis_name="s")


@functools.partial(
    pl.kernel,
    out_type=jax.ShapeDtypeStruct((NC * NS * N,), jnp.float32),
    mesh=_sc_mesh,
    scratch_types=[
        pltpu.VMEM((EPT,), jnp.int32),
        pltpu.VMEM((N,), jnp.float32),
    ],
    compiler_params=pltpu.CompilerParams(needs_layout_passes=False),
)
def _degree_kernel(eidx_hbm, out_hbm, ibuf, hist):
    c = lax.axis_index("c")
    s = lax.axis_index("s")
    base = pl.multiple_of(c * E + s * EPT, 8)
    pltpu.sync_copy(eidx_hbm.at[pl.ds(base, EPT)], ibuf)

    zeros = jnp.zeros((L,), jnp.float32)

    def zero_body(k, carry):
        hist[pl.ds(k * L, L)] = zeros
        return carry

    lax.fori_loop(0, N // L, zero_body, 0)

    ones = jnp.ones((L,), jnp.float32)
    full = jnp.ones((L,), jnp.bool_)

    def acc_body(k, carry):
        for u in range(5):
            idx = ibuf[pl.ds((k * 5 + u) * L, L)]
            plsc.addupdate_scatter(hist, [idx], ones, mask=full)
        return carry

    lax.fori_loop(0, EPT // (5 * L), acc_body, 0)
    out_base = pl.multiple_of((c * NS + s) * N, 8)
    pltpu.sync_copy(hist, out_hbm.at[pl.ds(out_base, N)])


def _norm_body(dp_ref, n_ref):
    deg = jnp.sum(dp_ref[...], axis=1)
    n_ref[...] = lax.rsqrt(jnp.clip(deg, 1.0, None))[:, :, None]


def _norm(partials):
    return pl.pallas_call(
        _norm_body,
        out_shape=jax.ShapeDtypeStruct((NC, N, 1), jnp.float32),
        grid=(1,),
        in_specs=[pl.BlockSpec((NC, NS, N), lambda i: (0, 0, 0))],
        out_specs=pl.BlockSpec((NC, N, 1), lambda i: (0, 0, 0)),
    )(partials)


def _scale_body(x_ref, h_ref, ns_ref, y_ref):
    ns = ns_ref[...]
    y_ref[0] = x_ref[...] * ns
    y_ref[1] = h_ref[...] * ns


_BLK = 2000


def _scale(inputs, hidden_state, norm_src):
    return pl.pallas_call(
        _scale_body,
        out_shape=jax.ShapeDtypeStruct((NC, N, D), jnp.float32),
        grid=(N // _BLK,),
        in_specs=[
            pl.BlockSpec((_BLK, D), lambda i: (i, 0)),
            pl.BlockSpec((_BLK, D), lambda i: (i, 0)),
            pl.BlockSpec((_BLK, 1), lambda i: (i, 0)),
        ],
        out_specs=pl.BlockSpec((NC, _BLK, D), lambda i: (0, i, 0)),
    )(inputs, hidden_state, norm_src)


MAXCH = NFULL + 1


@functools.partial(
    pl.kernel,
    out_type=jax.ShapeDtypeStruct((NC, N, D), jnp.float32),
    mesh=_sc_mesh,
    scratch_types=[
        pltpu.VMEM_SHARED((NPAD, D), jnp.float32),
        pltpu.VMEM((MAXCH * CHUNK,), jnp.int32),
        pltpu.VMEM((CHUNK,), jnp.int32),
        pltpu.VMEM((CHUNK,), jnp.int32),
        pltpu.VMEM((CHUNK, D), jnp.float32),
        pltpu.VMEM((CHUNK, D), jnp.float32),
        pltpu.VMEM((ZR, D), jnp.float32),
        pltpu.SemaphoreType.DMA,
        pltpu.SemaphoreType.DMA,
        pltpu.SemaphoreType.DMA,
        pltpu.SemaphoreType.DMA,
        pltpu.SemaphoreType.DMA,
    ],
)
def _scatter_kernel(eidx_hbm, y_hbm, out_hbm, acc, bsidx, didx0, didx1,
                    msg0, msg1, zbuf, gsem0, gsem1, dsem0, dsem1, zsem):
    c = lax.axis_index("c")
    s = lax.axis_index("s")

    ncs = NFULL + jnp.where(s < EXTRA, 1, 0)
    cs = s * NFULL + jnp.minimum(s, EXTRA)

    pltpu.async_copy(
        eidx_hbm.at[pl.ds(pl.multiple_of(cs * CHUNK, 8), MAXCH * CHUNK)],
        bsidx, gsem0)

    zeros = jnp.zeros((L,), jnp.float32)

    def zero_body(k, carry):
        zbuf[k // (D // L), pl.ds((k % (D // L)) * L, L)] = zeros
        return carry

    lax.fori_loop(0, ZR * D // L, zero_body, 0)
    pltpu.make_async_copy(
        eidx_hbm.at[pl.ds(0, MAXCH * CHUNK)], bsidx, gsem0).wait()

    off = (c * N).astype(jnp.int32) if hasattr(c, "astype") else c * N

    def shift_body(k, carry):
        for u in range(4):
            sl = pl.ds((k * 4 + u) * L, L)
            bsidx[sl] = bsidx[sl] + off
        return carry

    lax.fori_loop(0, MAXCH * CHUNK // (4 * L), shift_body, 0)
    for i in range(RPT // ZR):
        row0 = pl.multiple_of(s * RPT + i * ZR, 8)
        pltpu.async_copy(zbuf, acc.at[pl.ds(row0, ZR), :], zsem)

    dbase = E + cs * CHUNK

    def fetch(j, didx, msg, gsem, dsem):
        pltpu.async_copy(
            eidx_hbm.at[pl.ds(pl.multiple_of(dbase + j * CHUNK, 8), CHUNK)],
            didx, dsem)
        pltpu.async_copy(
            y_hbm.at[bsidx.at[pl.ds(j * CHUNK, CHUNK)]], msg, gsem)

    def drain(didx, msg, gsem, dsem):
        pltpu.make_async_copy(eidx_hbm.at[pl.ds(0, CHUNK)], didx, dsem).wait()
        pltpu.make_async_copy(y_hbm.at[pl.ds(0, CHUNK), :], msg, gsem).wait()

    fetch(0, didx0, msg0, gsem0, dsem0)
    fetch(1, didx1, msg1, gsem1, dsem1)
    for i in range(RPT // ZR):
        pltpu.make_async_copy(zbuf, acc.at[pl.ds(0, ZR), :], zsem).wait()
    plsc.subcore_barrier()

    def body(jj, carry):
        a = 2 * jj

        drain(didx0, msg0, gsem0, dsem0)
        pltpu.sync_copy(msg0, acc.at[didx0], add=True)

        @pl.when(a + 2 < ncs)
        def _():
            fetch(a + 2, didx0, msg0, gsem0, dsem0)

        drain(didx1, msg1, gsem1, dsem1)
        pltpu.sync_copy(msg1, acc.at[didx1], add=True)

        @pl.when(a + 3 < ncs)
        def _():
            fetch(a + 3, didx1, msg1, gsem1, dsem1)

        return carry

    lax.fori_loop(0, NFULL // 2, body, 0)

    @pl.when(s < EXTRA)
    def _():
        drain(didx0, msg0, gsem0, dsem0)
        pltpu.sync_copy(msg0, acc.at[didx0], add=True)

    plsc.subcore_barrier()

    row0 = pl.multiple_of(s * RPT, 8)

    @pl.when(s < NS - 1)
    def _():
        pltpu.sync_copy(acc.at[pl.ds(row0, RPT), :],
                        out_hbm.at[c, pl.ds(row0, RPT), :])

    @pl.when(s == NS - 1)
    def _():
        pltpu.sync_copy(acc.at[pl.ds(row0, TAIL), :],
                        out_hbm.at[c, pl.ds(row0, TAIL), :])


def _matmul_body(a_ref, w_ref, b_ref, nd_ref, o_ref):
    r = jnp.dot(a_ref[0], w_ref[:D, :], preferred_element_type=jnp.float32)
    r += jnp.dot(a_ref[1], w_ref[D:, :], preferred_element_type=jnp.float32)
    o_ref[...] = r * nd_ref[...] + b_ref[...]


def _matmul(agg2, W, b2, norm_dst):
    return pl.pallas_call(
        _matmul_body,
        out_shape=jax.ShapeDtypeStruct((N, DOUT), jnp.float32),
        grid=(N // _BLK,),
        in_specs=[
            pl.BlockSpec((NC, _BLK, D), lambda i: (0, i, 0)),
            pl.BlockSpec((DOUT, DOUT), lambda i: (0, 0)),
            pl.BlockSpec((1, DOUT), lambda i: (0, 0)),
            pl.BlockSpec((_BLK, 1), lambda i: (i, 0)),
        ],
        out_specs=pl.BlockSpec((_BLK, DOUT), lambda i: (i, 0)),
    )(agg2, W, b2, norm_dst)


def kernel(edge_index, inputs, hidden_state, W, b):
    eix = edge_index.reshape(2 * E)
    partials = _degree_kernel(eix).reshape(NC, NS, N)
    norms = _norm(partials)
    y2 = _scale(inputs, hidden_state, norms[0])
    agg2 = _scatter_kernel(eix, y2.reshape(NC * N, D))
    return _matmul(agg2, W, b.reshape(1, DOUT), norms[1])


# speedup vs baseline: 1.0514x; 1.0514x over previous
import functools

import jax
import jax.numpy as jnp
from jax import lax
from jax.experimental import pallas as pl
from jax.experimental.pallas import tpu as pltpu
from jax.experimental.pallas import tpu_sc as plsc

N = 10000
E = 160000
D = 128
DOUT = 256
NC = 2
NS = 16
L = 16

CHUNK = 128
NCHUNKS = E // CHUNK
NFULL = NCHUNKS // NS
EXTRA = NCHUNKS % NS
EPT = E // NS

NPAD = 10240
RPT = NPAD // NS
ZR = 32
TAIL = N - (NS - 1) * RPT

_sc_mesh = plsc.VectorSubcoreMesh(core_axis_name="c", subcore_axis_name="s")


@functools.partial(
    pl.kernel,
    out_type=jax.ShapeDtypeStruct((NC * NS * N,), jnp.float32),
    mesh=_sc_mesh,
    scratch_types=[
        pltpu.VMEM((EPT,), jnp.int32),
        pltpu.VMEM((N,), jnp.float32),
    ],
    compiler_params=pltpu.CompilerParams(needs_layout_passes=False),
)
def _degree_kernel(eidx_hbm, out_hbm, ibuf, hist):
    c = lax.axis_index("c")
    s = lax.axis_index("s")
    base = pl.multiple_of(c * (2 * E) + s * EPT, 8)
    pltpu.sync_copy(eidx_hbm.at[pl.ds(base, EPT)], ibuf)

    zeros = jnp.zeros((L,), jnp.float32)

    def zero_body(k, carry):
        hist[pl.ds(k * L, L)] = zeros
        return carry

    lax.fori_loop(0, N // L, zero_body, 0)

    ones = jnp.ones((L,), jnp.float32)
    full = jnp.ones((L,), jnp.bool_)

    def acc_body(k, carry):
        for u in range(5):
            idx = ibuf[pl.ds((k * 5 + u) * L, L)]
            plsc.addupdate_scatter(hist, [idx], ones, mask=full)
        return carry

    lax.fori_loop(0, EPT // (5 * L), acc_body, 0)
    out_base = pl.multiple_of((c * NS + s) * N, 8)
    pltpu.sync_copy(hist, out_hbm.at[pl.ds(out_base, N)])


def _scale_body(x_ref, h_ref, dp_ref, y_ref):
    deg = jnp.sum(dp_ref[...], axis=1)
    ns = lax.rsqrt(jnp.clip(deg, 1.0, None))
    y_ref[0] = x_ref[...] * ns[:, None]
    y_ref[1] = h_ref[...] * ns[:, None]


_BLK = 2000


def _scale(inputs, hidden_state, deg_partials_src):
    return pl.pallas_call(
        _scale_body,
        out_shape=jax.ShapeDtypeStruct((NC, N, D), jnp.float32),
        grid=(N // _BLK,),
        in_specs=[
            pl.BlockSpec((_BLK, D), lambda i: (i, 0)),
            pl.BlockSpec((_BLK, D), lambda i: (i, 0)),
            pl.BlockSpec((_BLK, NS), lambda i: (i, 0)),
        ],
        out_specs=pl.BlockSpec((NC, _BLK, D), lambda i: (0, i, 0)),
    )(inputs, hidden_state, deg_partials_src)


MAXCH = NFULL + 1


@functools.partial(
    pl.kernel,
    out_type=jax.ShapeDtypeStruct((NC, N, D), jnp.float32),
    mesh=_sc_mesh,
    scratch_types=[
        pltpu.VMEM_SHARED((NPAD, D), jnp.float32),
        pltpu.VMEM((MAXCH * CHUNK,), jnp.int32),
        pltpu.VMEM((CHUNK,), jnp.int32),
        pltpu.VMEM((CHUNK,), jnp.int32),
        pltpu.VMEM((CHUNK, D), jnp.float32),
        pltpu.VMEM((CHUNK, D), jnp.float32),
        pltpu.VMEM((ZR, D), jnp.float32),
        pltpu.SemaphoreType.DMA,
        pltpu.SemaphoreType.DMA,
        pltpu.SemaphoreType.DMA,
        pltpu.SemaphoreType.DMA,
        pltpu.SemaphoreType.DMA,
    ],
)
def _scatter_kernel(eidx_hbm, y_hbm, out_hbm, acc, bsidx, didx0, didx1,
                    msg0, msg1, zbuf, gsem0, gsem1, dsem0, dsem1, zsem):
    c = lax.axis_index("c")
    s = lax.axis_index("s")

    ncs = NFULL + jnp.where(s < EXTRA, 1, 0)
    cs = s * NFULL + jnp.minimum(s, EXTRA)

    pltpu.async_copy(
        eidx_hbm.at[pl.ds(pl.multiple_of(c * E + cs * CHUNK, 8),
                          MAXCH * CHUNK)],
        bsidx, gsem0)

    zeros = jnp.zeros((L,), jnp.float32)

    def zero_body(k, carry):
        zbuf[k // (D // L), pl.ds((k % (D // L)) * L, L)] = zeros
        return carry

    lax.fori_loop(0, ZR * D // L, zero_body, 0)
    pltpu.make_async_copy(
        eidx_hbm.at[pl.ds(0, MAXCH * CHUNK)], bsidx, gsem0).wait()
    for i in range(RPT // ZR):
        row0 = pl.multiple_of(s * RPT + i * ZR, 8)
        pltpu.async_copy(zbuf, acc.at[pl.ds(row0, ZR), :], zsem)

    dbase = 2 * E + cs * CHUNK

    def fetch(j, didx, msg, gsem, dsem):
        pltpu.async_copy(
            eidx_hbm.at[pl.ds(pl.multiple_of(dbase + j * CHUNK, 8), CHUNK)],
            didx, dsem)
        pltpu.async_copy(
            y_hbm.at[bsidx.at[pl.ds(j * CHUNK, CHUNK)]], msg, gsem)

    def drain(didx, msg, gsem, dsem):
        pltpu.make_async_copy(eidx_hbm.at[pl.ds(0, CHUNK)], didx, dsem).wait()
        pltpu.make_async_copy(y_hbm.at[pl.ds(0, CHUNK), :], msg, gsem).wait()

    fetch(0, didx0, msg0, gsem0, dsem0)
    fetch(1, didx1, msg1, gsem1, dsem1)
    for i in range(RPT // ZR):
        pltpu.make_async_copy(zbuf, acc.at[pl.ds(0, ZR), :], zsem).wait()
    plsc.subcore_barrier()

    def body(jj, carry):
        a = 2 * jj

        drain(didx0, msg0, gsem0, dsem0)
        pltpu.sync_copy(msg0, acc.at[didx0], add=True)

        @pl.when(a + 2 < ncs)
        def _():
            fetch(a + 2, didx0, msg0, gsem0, dsem0)

        drain(didx1, msg1, gsem1, dsem1)
        pltpu.sync_copy(msg1, acc.at[didx1], add=True)

        @pl.when(a + 3 < ncs)
        def _():
            fetch(a + 3, didx1, msg1, gsem1, dsem1)

        return carry

    lax.fori_loop(0, NFULL // 2, body, 0)

    @pl.when(s < EXTRA)
    def _():
        drain(didx0, msg0, gsem0, dsem0)
        pltpu.sync_copy(msg0, acc.at[didx0], add=True)

    plsc.subcore_barrier()

    row0 = pl.multiple_of(s * RPT, 8)

    @pl.when(s < NS - 1)
    def _():
        pltpu.sync_copy(acc.at[pl.ds(row0, RPT), :],
                        out_hbm.at[c, pl.ds(row0, RPT), :])

    @pl.when(s == NS - 1)
    def _():
        pltpu.sync_copy(acc.at[pl.ds(row0, TAIL), :],
                        out_hbm.at[c, pl.ds(row0, TAIL), :])


def _matmul_body(a_ref, w_ref, b_ref, dp_ref, o_ref):
    r = jnp.dot(a_ref[0], w_ref[:D, :], preferred_element_type=jnp.float32)
    r += jnp.dot(a_ref[1], w_ref[D:, :], preferred_element_type=jnp.float32)
    deg = jnp.sum(dp_ref[...], axis=1)
    nd = lax.rsqrt(jnp.clip(deg, 1.0, None))
    o_ref[...] = r * nd[:, None] + b_ref[...]


def _matmul(agg2, W, b2, deg_partials_dst):
    return pl.pallas_call(
        _matmul_body,
        out_shape=jax.ShapeDtypeStruct((N, DOUT), jnp.float32),
        grid=(N // _BLK,),
        in_specs=[
            pl.BlockSpec((NC, _BLK, D), lambda i: (0, i, 0)),
            pl.BlockSpec((DOUT, DOUT), lambda i: (0, 0)),
            pl.BlockSpec((1, DOUT), lambda i: (0, 0)),
            pl.BlockSpec((_BLK, NS), lambda i: (i, 0)),
        ],
        out_specs=pl.BlockSpec((_BLK, DOUT), lambda i: (i, 0)),
    )(agg2, W, b2, deg_partials_dst)


def kernel(edge_index, inputs, hidden_state, W, b):
    src = edge_index[0]
    dst = edge_index[1]
    eidx = jnp.concatenate([src, src + N, dst])
    partials = _degree_kernel(eidx).reshape(NC, NS, N)
    dp_src = partials[0].T
    dp_dst = partials[1].T
    y2 = _scale(inputs, hidden_state, dp_src)
    agg2 = _scatter_kernel(eidx, y2.reshape(NC * N, D))
    return _matmul(agg2, W, b.reshape(1, DOUT), dp_dst)


# speedup vs baseline: 1.0882x; 1.0350x over previous
import functools

import jax
import jax.numpy as jnp
from jax import lax
from jax.experimental import pallas as pl
from jax.experimental.pallas import tpu as pltpu
from jax.experimental.pallas import tpu_sc as plsc

N = 10000
E = 160000
D = 128
DOUT = 256
NC = 2
NS = 16
L = 16

CHUNK = 128
NCHUNKS = E // CHUNK
NFULL = NCHUNKS // NS
EXTRA = NCHUNKS % NS
EPT = E // NS

NPAD = 10240
RPT = NPAD // NS
ZR = 32
TAIL = N - (NS - 1) * RPT

_sc_mesh = plsc.VectorSubcoreMesh(core_axis_name="c", subcore_axis_name="s")


@functools.partial(
    pl.kernel,
    out_type=jax.ShapeDtypeStruct((NC * NS * N,), jnp.float32),
    mesh=_sc_mesh,
    scratch_types=[
        pltpu.VMEM((EPT,), jnp.int32),
        pltpu.VMEM((N,), jnp.float32),
    ],
    compiler_params=pltpu.CompilerParams(needs_layout_passes=False),
)
def _degree_kernel(eidx_hbm, out_hbm, ibuf, hist):
    c = lax.axis_index("c")
    s = lax.axis_index("s")
    base = pl.multiple_of(c * E + s * EPT, 8)
    pltpu.sync_copy(eidx_hbm.at[pl.ds(base, EPT)], ibuf)

    zeros = jnp.zeros((L,), jnp.float32)

    def zero_body(k, carry):
        hist[pl.ds(k * L, L)] = zeros
        return carry

    lax.fori_loop(0, N // L, zero_body, 0)

    ones = jnp.ones((L,), jnp.float32)
    full = jnp.ones((L,), jnp.bool_)

    def acc_body(k, carry):
        for u in range(5):
            idx = ibuf[pl.ds((k * 5 + u) * L, L)]
            plsc.addupdate_scatter(hist, [idx], ones, mask=full)
        return carry

    lax.fori_loop(0, EPT // (5 * L), acc_body, 0)
    out_base = pl.multiple_of((c * NS + s) * N, 8)
    pltpu.sync_copy(hist, out_hbm.at[pl.ds(out_base, N)])


def _scale_body(x_ref, h_ref, dp_ref, y_ref):
    deg = jnp.sum(dp_ref[...], axis=1)
    ns = lax.rsqrt(jnp.clip(deg, 1.0, None))
    y_ref[0] = x_ref[...] * ns[:, None]
    y_ref[1] = h_ref[...] * ns[:, None]


_BLK = 2000


def _scale(inputs, hidden_state, deg_partials_src):
    return pl.pallas_call(
        _scale_body,
        out_shape=jax.ShapeDtypeStruct((NC, N, D), jnp.float32),
        grid=(N // _BLK,),
        in_specs=[
            pl.BlockSpec((_BLK, D), lambda i: (i, 0)),
            pl.BlockSpec((_BLK, D), lambda i: (i, 0)),
            pl.BlockSpec((_BLK, NS), lambda i: (i, 0)),
        ],
        out_specs=pl.BlockSpec((NC, _BLK, D), lambda i: (0, i, 0)),
    )(inputs, hidden_state, deg_partials_src)


MAXCH = NFULL + 1


@functools.partial(
    pl.kernel,
    out_type=jax.ShapeDtypeStruct((NC, N, D), jnp.float32),
    mesh=_sc_mesh,
    scratch_types=[
        pltpu.VMEM_SHARED((NPAD, D), jnp.float32),
        pltpu.VMEM((MAXCH * CHUNK,), jnp.int32),
        pltpu.VMEM((CHUNK,), jnp.int32),
        pltpu.VMEM((CHUNK,), jnp.int32),
        pltpu.VMEM((CHUNK, D), jnp.float32),
        pltpu.VMEM((CHUNK, D), jnp.float32),
        pltpu.VMEM((ZR, D), jnp.float32),
        pltpu.SemaphoreType.DMA,
        pltpu.SemaphoreType.DMA,
        pltpu.SemaphoreType.DMA,
        pltpu.SemaphoreType.DMA,
        pltpu.SemaphoreType.DMA,
    ],
)
def _scatter_kernel(eidx_hbm, y_hbm, out_hbm, acc, bsidx, didx0, didx1,
                    msg0, msg1, zbuf, gsem0, gsem1, dsem0, dsem1, zsem):
    c = lax.axis_index("c")
    s = lax.axis_index("s")

    ncs = NFULL + jnp.where(s < EXTRA, 1, 0)
    cs = s * NFULL + jnp.minimum(s, EXTRA)

    pltpu.async_copy(
        eidx_hbm.at[pl.ds(pl.multiple_of(cs * CHUNK, 8), MAXCH * CHUNK)],
        bsidx, gsem0)

    zeros = jnp.zeros((L,), jnp.float32)

    def zero_body(k, carry):
        zbuf[k // (D // L), pl.ds((k % (D // L)) * L, L)] = zeros
        return carry

    lax.fori_loop(0, ZR * D // L, zero_body, 0)
    pltpu.make_async_copy(
        eidx_hbm.at[pl.ds(0, MAXCH * CHUNK)], bsidx, gsem0).wait()

    @pl.when(c == 1)
    def _():
        def shift_head(k, carry):
            sl = pl.ds(k * L, L)
            bsidx[sl] = bsidx[sl] + N
            return carry

        lax.fori_loop(0, 2 * CHUNK // L, shift_head, 0)

    dbase = E + cs * CHUNK

    def fetch(j, didx, msg, gsem, dsem):
        pltpu.async_copy(
            eidx_hbm.at[pl.ds(pl.multiple_of(dbase + j * CHUNK, 8), CHUNK)],
            didx, dsem)
        pltpu.async_copy(
            y_hbm.at[bsidx.at[pl.ds(j * CHUNK, CHUNK)]], msg, gsem)

    def drain(didx, msg, gsem, dsem):
        pltpu.make_async_copy(eidx_hbm.at[pl.ds(0, CHUNK)], didx, dsem).wait()
        pltpu.make_async_copy(y_hbm.at[pl.ds(0, CHUNK), :], msg, gsem).wait()

    fetch(0, didx0, msg0, gsem0, dsem0)
    fetch(1, didx1, msg1, gsem1, dsem1)

    for i in range(RPT // ZR):
        row0 = pl.multiple_of(s * RPT + i * ZR, 8)
        pltpu.async_copy(zbuf, acc.at[pl.ds(row0, ZR), :], zsem)

    @pl.when(c == 1)
    def _():
        def shift_tail(k, carry):
            for u in range(4):
                sl = pl.ds((2 * CHUNK // L + k * 4 + u) * L, L)
                bsidx[sl] = bsidx[sl] + N
            return carry

        lax.fori_loop(0, (MAXCH - 2) * CHUNK // (4 * L), shift_tail, 0)

    for i in range(RPT // ZR):
        pltpu.make_async_copy(zbuf, acc.at[pl.ds(0, ZR), :], zsem).wait()
    plsc.subcore_barrier()

    def body(jj, carry):
        a = 2 * jj

        drain(didx0, msg0, gsem0, dsem0)
        pltpu.sync_copy(msg0, acc.at[didx0], add=True)

        @pl.when(a + 2 < ncs)
        def _():
            fetch(a + 2, didx0, msg0, gsem0, dsem0)

        drain(didx1, msg1, gsem1, dsem1)
        pltpu.sync_copy(msg1, acc.at[didx1], add=True)

        @pl.when(a + 3 < ncs)
        def _():
            fetch(a + 3, didx1, msg1, gsem1, dsem1)

        return carry

    lax.fori_loop(0, NFULL // 2, body, 0)

    @pl.when(s < EXTRA)
    def _():
        drain(didx0, msg0, gsem0, dsem0)
        pltpu.sync_copy(msg0, acc.at[didx0], add=True)

    plsc.subcore_barrier()

    row0 = pl.multiple_of(s * RPT, 8)

    @pl.when(s < NS - 1)
    def _():
        pltpu.sync_copy(acc.at[pl.ds(row0, RPT), :],
                        out_hbm.at[c, pl.ds(row0, RPT), :])

    @pl.when(s == NS - 1)
    def _():
        pltpu.sync_copy(acc.at[pl.ds(row0, TAIL), :],
                        out_hbm.at[c, pl.ds(row0, TAIL), :])


def _matmul_body(a_ref, w_ref, b_ref, dp_ref, o_ref):
    r = jnp.dot(a_ref[0], w_ref[:D, :], preferred_element_type=jnp.float32)
    r += jnp.dot(a_ref[1], w_ref[D:, :], preferred_element_type=jnp.float32)
    deg = jnp.sum(dp_ref[...], axis=1)
    nd = lax.rsqrt(jnp.clip(deg, 1.0, None))
    o_ref[...] = r * nd[:, None] + b_ref[...]


def _matmul(agg2, W, b2, deg_partials_dst):
    return pl.pallas_call(
        _matmul_body,
        out_shape=jax.ShapeDtypeStruct((N, DOUT), jnp.float32),
        grid=(N // _BLK,),
        in_specs=[
            pl.BlockSpec((NC, _BLK, D), lambda i: (0, i, 0)),
            pl.BlockSpec((DOUT, DOUT), lambda i: (0, 0)),
            pl.BlockSpec((1, DOUT), lambda i: (0, 0)),
            pl.BlockSpec((_BLK, NS), lambda i: (i, 0)),
        ],
        out_specs=pl.BlockSpec((_BLK, DOUT), lambda i: (i, 0)),
    )(agg2, W, b2, deg_partials_dst)


def kernel(edge_index, inputs, hidden_state, W, b):
    eix = edge_index.reshape(2 * E)
    partials = _degree_kernel(eix).reshape(NC, NS, N)
    dp_src = partials[0].T
    dp_dst = partials[1].T
    y2 = _scale(inputs, hidden_state, dp_src)
    agg2 = _scatter_kernel(eix, y2.reshape(NC * N, D))
    return _matmul(agg2, W, b.reshape(1, DOUT), dp_dst)
